# baseline (speedup 1.0000x reference)
import jax
import jax.numpy as jnp
from jax.experimental import pallas as pl
from jax.experimental.pallas import tpu as pltpu


def _mix_kernel(index_ref, lam_ref, x_ref, xs_ref, o_ref):
    l = lam_ref[0]
    o_ref[...] = l * x_ref[...] + (1.0 - l) * xs_ref[...]


def kernel(x, lam, index):
    B = x.shape[0]
    xf = x.reshape(B, 1, -1)
    C = xf.shape[2]
    out = pl.pallas_call(
        _mix_kernel,
        grid_spec=pltpu.PrefetchScalarGridSpec(
            num_scalar_prefetch=2,
            grid=(B,),
            in_specs=[
                pl.BlockSpec((1, 1, C), lambda i, idx_ref, lam_ref: (i, 0, 0)),
                pl.BlockSpec((1, 1, C), lambda i, idx_ref, lam_ref: (idx_ref[i], 0, 0)),
            ],
            out_specs=pl.BlockSpec((1, 1, C), lambda i, idx_ref, lam_ref: (i, 0, 0)),
        ),
        out_shape=jax.ShapeDtypeStruct((B, 1, C), x.dtype),
    )(index.astype(jnp.int32), lam, xf, xf)
    return out.reshape(x.shape)


# speedup vs baseline: 1.3673x; 1.3673x over previous
import jax
import jax.numpy as jnp
from jax.experimental import pallas as pl
from jax.experimental.pallas import tpu as pltpu


def _mix_kernel(index_ref, lam_ref, x_ref, xs_ref, o_ref):
    l = lam_ref[0]
    o_ref[...] = l * x_ref[...] + (1.0 - l) * xs_ref[...]


def kernel(x, lam, index):
    B = x.shape[0]
    C = x.size // B
    L = 512
    S = C // L
    xf = x.reshape(B, S, L)
    out = pl.pallas_call(
        _mix_kernel,
        grid_spec=pltpu.PrefetchScalarGridSpec(
            num_scalar_prefetch=2,
            grid=(B,),
            in_specs=[
                pl.BlockSpec((1, S, L), lambda i, idx_ref, lam_ref: (i, 0, 0)),
                pl.BlockSpec((1, S, L), lambda i, idx_ref, lam_ref: (idx_ref[i], 0, 0)),
            ],
            out_specs=pl.BlockSpec((1, S, L), lambda i, idx_ref, lam_ref: (i, 0, 0)),
        ),
        out_shape=jax.ShapeDtypeStruct((B, S, L), x.dtype),
    )(index.astype(jnp.int32), lam, xf, xf)
    return out.reshape(x.shape)


# speedup vs baseline: 1.4421x; 1.0547x over previous
import jax
import jax.numpy as jnp
from jax.experimental import pallas as pl
from jax.experimental.pallas import tpu as pltpu

_G = 8


def _mix_kernel(idx_ref, lam_ref, xd_ref, xh_ref, o_ref, gbuf, sems):
    i = pl.program_id(0)
    nsteps = pl.num_programs(0)

    def start_copies(step, slot):
        for g in range(_G):
            row = idx_ref[step * _G + g]
            pltpu.make_async_copy(
                xh_ref.at[row],
                gbuf.at[slot, g],
                sems.at[slot, g],
            ).start()

    def wait_copies(slot):
        for g in range(_G):
            pltpu.make_async_copy(
                gbuf.at[slot, g],
                gbuf.at[slot, g],
                sems.at[slot, g],
            ).wait()

    @pl.when(i == 0)
    def _():
        start_copies(0, 0)

    @pl.when(i + 1 < nsteps)
    def _():
        start_copies(i + 1, (i + 1) % 2)

    wait_copies(i % 2)
    l = lam_ref[0]
    o_ref[...] = l * xd_ref[...] + (1.0 - l) * gbuf[i % 2]


def kernel(x, lam, index):
    B = x.shape[0]
    C = x.size // B
    L = 512
    S = C // L
    xf = x.reshape(B, S, L)
    out = pl.pallas_call(
        _mix_kernel,
        grid_spec=pltpu.PrefetchScalarGridSpec(
            num_scalar_prefetch=2,
            grid=(B // _G,),
            in_specs=[
                pl.BlockSpec((_G, S, L), lambda i, idx_ref, lam_ref: (i, 0, 0)),
                pl.BlockSpec(memory_space=pl.ANY),
            ],
            out_specs=pl.BlockSpec((_G, S, L), lambda i, idx_ref, lam_ref: (i, 0, 0)),
            scratch_shapes=[
                pltpu.VMEM((2, _G, S, L), jnp.float32),
                pltpu.SemaphoreType.DMA((2, _G)),
            ],
        ),
        out_shape=jax.ShapeDtypeStruct((B, S, L), x.dtype),
    )(index.astype(jnp.int32), lam, xf, xf)
    return out.reshape(x.shape)
